# baseline (speedup 1.0000x reference)
import functools

import jax
import jax.numpy as jnp
from jax import lax
from jax.experimental import pallas as pl
from jax.experimental.pallas import tpu as pltpu
from jax.experimental.pallas import tpu_sc as plsc

NC = 2
NS = 16
NW = NC * NS
LANES = 16
NBUF = 4
PREF = 2


def _make_kernel(B, L, V, D, C, interpret=False):
    BC = B // NW
    assert BC % LANES == 0 and BC <= 128 and L == C and D % LANES == 0
    assert L % NBUF == 0

    mesh = plsc.VectorSubcoreMesh(core_axis_name="c", subcore_axis_name="s")

    @functools.partial(
        pl.kernel,
        out_type=jax.ShapeDtypeStruct((L, D, B), jnp.float32),
        mesh=mesh,
        compiler_params=pltpu.CompilerParams(
            use_tc_tiling_on_sc=False, needs_layout_passes=False),
        interpret=interpret,
        scratch_types=[
            pltpu.VMEM((C, BC), jnp.int32),
            pltpu.VMEM((NBUF, BC, D), jnp.float32),
            pltpu.VMEM((NBUF, D, BC), jnp.float32),
            pltpu.VMEM((C, D), jnp.float32),
            pltpu.VMEM((2 * LANES,), jnp.float32),
            [pltpu.SemaphoreType.DMA] * NBUF,
            [pltpu.SemaphoreType.DMA] * NBUF,
        ],
    )
    def k(xT_hbm, tok_hbm, pos_hbm, out_hbm,
          idx_v, rows_v, outT_v, pos_v, ptmp_v, gsems, wsems):
        wid = lax.axis_index("s") * NC + lax.axis_index("c")
        b0 = wid * BC

        pltpu.sync_copy(xT_hbm.at[:, pl.ds(b0, BC)], idx_v)
        pltpu.sync_copy(pos_hbm, pos_v)

        def gather(lp, p):
            return pltpu.make_async_copy(
                tok_hbm.at[idx_v.at[lp]], rows_v.at[p], gsems[p])

        def writeback(lp, p):
            return pltpu.make_async_copy(
                outT_v.at[p], out_hbm.at[lp, :, pl.ds(b0, BC)], wsems[p])

        for j in range(PREF):
            gather(j, j).start()

        rowbase = lax.broadcasted_iota(jnp.int32, (LANES,), 0)

        def step(l2, j):
            lp = NBUF * l2 + j
            p = j
            q = (j + PREF) % NBUF

            gather(lp, p).wait()

            def dstep(dc, carry):
                ptmp_v[pl.ds(LANES, LANES)] = pos_v[lp, pl.ds(dc * LANES, LANES)]
                for dj in range(LANES):
                    d = dc * LANES + dj
                    pv = plsc.load_gather(
                        ptmp_v, [jnp.full((LANES,), LANES + dj, jnp.int32)])
                    for bs in range(BC // LANES):
                        col = plsc.load_gather(
                            rows_v,
                            [jnp.full((LANES,), p, jnp.int32),
                             rowbase + bs * LANES,
                             jnp.full((LANES,), d, jnp.int32)])
                        outT_v[p, d, pl.ds(bs * LANES, LANES)] = col + pv
                return carry

            lax.fori_loop(0, D // LANES, dstep, 0)

            writeback(lp, p).start()

            if j < PREF:
                @pl.when(l2 >= 1)
                def _():
                    writeback(lp - PREF, q).wait()

                gather(lp + PREF, q).start()
            else:
                writeback(lp - PREF, q).wait()

                @pl.when(l2 < L // NBUF - 1)
                def _():
                    gather(lp + PREF, q).start()

        def round4(l2, carry):
            for j in range(NBUF):
                step(l2, j)
            return carry

        lax.fori_loop(0, L // NBUF, round4, 0)

        for lp in range(L - PREF, L):
            writeback(lp, lp % NBUF).wait()

    return k


def kernel(x, tok_table, pos_table):
    B, L = x.shape
    V, D = tok_table.shape
    C = pos_table.shape[0]
    k = _make_kernel(B, L, V, D, C)
    out3 = k(x.T, tok_table, pos_table)
    return jnp.transpose(out3, (2, 0, 1))


# speedup vs baseline: 1.1182x; 1.1182x over previous
import functools

import jax
import jax.numpy as jnp
from jax import lax
from jax.experimental import pallas as pl
from jax.experimental.pallas import tpu as pltpu
from jax.experimental.pallas import tpu_sc as plsc

NC = 2
NS = 16
NW = NC * NS
LANES = 16
NBUF = 4
PREF = 2
SUBL = 8
TL = 128


def _make_kernel(B, L, V, D, C, interpret=False):
    BC = B // NW
    DT = D // SUBL
    assert BC == TL and L == C and D % LANES == 0
    assert L % NBUF == 0

    mesh = plsc.VectorSubcoreMesh(core_axis_name="c", subcore_axis_name="s")

    @functools.partial(
        pl.kernel,
        out_type=jax.ShapeDtypeStruct((L, DT, NW, SUBL * TL), jnp.float32),
        mesh=mesh,
        compiler_params=pltpu.CompilerParams(
            use_tc_tiling_on_sc=False, needs_layout_passes=False),
        interpret=interpret,
        scratch_types=[
            pltpu.VMEM((C, BC), jnp.int32),
            pltpu.VMEM((NBUF, BC, D), jnp.float32),
            pltpu.VMEM((NBUF, DT, SUBL * TL), jnp.float32),
            pltpu.VMEM((C, D), jnp.float32),
            [pltpu.SemaphoreType.DMA] * NBUF,
            [pltpu.SemaphoreType.DMA] * NBUF,
        ],
    )
    def k(xT_hbm, tok_hbm, pos_hbm, out_hbm,
          idx_v, rows_v, outT_v, pos_v, gsems, wsems):
        wid = lax.axis_index("s") * NC + lax.axis_index("c")
        b0 = wid * BC

        pltpu.sync_copy(xT_hbm.at[:, pl.ds(b0, BC)], idx_v)
        pltpu.sync_copy(pos_hbm, pos_v)

        def gather(lp, p):
            return pltpu.make_async_copy(
                tok_hbm.at[idx_v.at[lp]], rows_v.at[p], gsems[p])

        def writeback(lp, p):
            return pltpu.make_async_copy(
                outT_v.at[p], out_hbm.at[lp, :, wid, :], wsems[p])

        for j in range(PREF):
            gather(j, j).start()

        iota = lax.broadcasted_iota(jnp.int32, (LANES,), 0)
        rb = [iota + bs * LANES for bs in range(BC // LANES)]
        csts = [jnp.full((LANES,), dj, jnp.int32) for dj in range(LANES)]

        def step(l2, j):
            lp = NBUF * l2 + j
            p = j
            q = (j + PREF) % NBUF

            gather(lp, p).wait()

            def dstep(dc, carry):
                pchunk = pos_v[lp, pl.ds(dc * LANES, LANES)]
                for dj in range(LANES):
                    d = dc * LANES + dj
                    pv = jnp.take(pchunk, csts[dj])
                    dt = 2 * dc + (dj // SUBL)
                    off = (dj % SUBL) * TL
                    colv = jnp.full((LANES,), d, jnp.int32)
                    for bs in range(BC // LANES):
                        col = plsc.load_gather(
                            rows_v,
                            [csts[p], rb[bs], colv])
                        outT_v[p, dt, pl.ds(off + bs * LANES, LANES)] = (
                            col + pv)
                return carry

            lax.fori_loop(0, D // LANES, dstep, 0)

            writeback(lp, p).start()

            if j < PREF:
                @pl.when(l2 >= 1)
                def _():
                    writeback(lp - PREF, q).wait()

                gather(lp + PREF, q).start()
            else:
                writeback(lp - PREF, q).wait()

                @pl.when(l2 < L // NBUF - 1)
                def _():
                    gather(lp + PREF, q).start()

        def round4(l2, carry):
            for j in range(NBUF):
                step(l2, j)
            return carry

        lax.fori_loop(0, L // NBUF, round4, 0)

        for lp in range(L - PREF, L):
            writeback(lp, lp % NBUF).wait()

    return k


def kernel(x, tok_table, pos_table):
    B, L = x.shape
    V, D = tok_table.shape
    C = pos_table.shape[0]
    k = _make_kernel(B, L, V, D, C)
    out4 = k(x.T, tok_table, pos_table)
    out5 = out4.reshape(L, D // SUBL, B // TL, SUBL, TL)
    out = jnp.transpose(out5, (2, 4, 0, 1, 3)).reshape(B, L, D)
    return out


# speedup vs baseline: 1.8576x; 1.6613x over previous
import functools

import jax
import jax.numpy as jnp
from jax import lax
from jax.experimental import pallas as pl
from jax.experimental.pallas import tpu as pltpu
from jax.experimental.pallas import tpu_sc as plsc

NC = 2
NS = 16
NW = NC * NS
LANES = 16
NBUF = 4
PREF = 2
SUBL = 8
TL = 128


def _make_kernel(B, L, V, D, C, interpret=False):
    BC = B // NW
    DT = D // SUBL
    assert BC == TL and L == C and D % LANES == 0
    assert L % NBUF == 0

    mesh = plsc.VectorSubcoreMesh(core_axis_name="c", subcore_axis_name="s")

    @functools.partial(
        pl.kernel,
        out_type=jax.ShapeDtypeStruct((L, DT, NW, SUBL * TL), jnp.float32),
        mesh=mesh,
        compiler_params=pltpu.CompilerParams(
            use_tc_tiling_on_sc=False, needs_layout_passes=False),
        interpret=interpret,
        scratch_types=[
            pltpu.VMEM((C, BC), jnp.int32),
            pltpu.VMEM((NBUF, BC, D), jnp.float32),
            pltpu.VMEM((NBUF, DT, SUBL * TL), jnp.float32),
            pltpu.VMEM((C, D), jnp.float32),
            pltpu.VMEM((2 * D,), jnp.float32),
            [pltpu.SemaphoreType.DMA] * NBUF,
            [pltpu.SemaphoreType.DMA] * NBUF,
        ],
    )
    def k(xT_hbm, tok_hbm, pos_hbm, out_hbm,
          idx_v, rows_v, outT_v, pos_v, posd_v, gsems, wsems):
        wid = lax.axis_index("s") * NC + lax.axis_index("c")
        b0 = wid * BC

        pltpu.sync_copy(xT_hbm.at[:, pl.ds(b0, BC)], idx_v)
        pltpu.sync_copy(pos_hbm, pos_v)

        def gather(lp, p):
            return pltpu.make_async_copy(
                tok_hbm.at[idx_v.at[lp]], rows_v.at[p], gsems[p])

        def writeback(lp, p):
            return pltpu.make_async_copy(
                outT_v.at[p], out_hbm.at[lp, :, wid, :], wsems[p])

        for j in range(PREF):
            gather(j, j).start()

        iota = lax.broadcasted_iota(jnp.int32, (LANES,), 0)
        rb = [iota + bs * LANES for bs in range(BC // LANES)]
        csts = [jnp.full((LANES,), v, jnp.int32) for v in range(NBUF)]

        def step(l2, j):
            lp = NBUF * l2 + j
            p = j
            q = (j + PREF) % NBUF

            gather(lp, p).wait()

            for c in range(D // LANES):
                pc = pos_v[lp, pl.ds(c * LANES, LANES)]
                posd_v[pl.ds(c * LANES, LANES)] = pc
                posd_v[pl.ds(D + c * LANES, LANES)] = pc

            def dgstep(dg, carry):
                for dj in range(LANES):
                    dstart = dg * LANES + dj
                    dvec = (dstart + iota) & (D - 1)
                    pv = posd_v[pl.ds(dstart, LANES)]
                    dtv = dvec >> 3
                    tbase = (dvec & 7) << 7
                    for bs in range(BC // LANES):
                        col = plsc.load_gather(
                            rows_v, [csts[p], rb[bs], dvec])
                        plsc.store_scatter(
                            outT_v,
                            [csts[p], dtv, tbase + rb[bs]],
                            col + pv)
                return carry

            lax.fori_loop(0, D // LANES, dgstep, 0)

            writeback(lp, p).start()

            if j < PREF:
                @pl.when(l2 >= 1)
                def _():
                    writeback(lp - PREF, q).wait()

                gather(lp + PREF, q).start()
            else:
                writeback(lp - PREF, q).wait()

                @pl.when(l2 < L // NBUF - 1)
                def _():
                    gather(lp + PREF, q).start()

        def round4(l2, carry):
            for j in range(NBUF):
                step(l2, j)
            return carry

        lax.fori_loop(0, L // NBUF, round4, 0)

        for lp in range(L - PREF, L):
            writeback(lp, lp % NBUF).wait()

    return k


def kernel(x, tok_table, pos_table):
    B, L = x.shape
    V, D = tok_table.shape
    C = pos_table.shape[0]
    k = _make_kernel(B, L, V, D, C)
    out4 = k(x.T, tok_table, pos_table)
    out5 = out4.reshape(L, D // SUBL, B // TL, SUBL, TL)
    out = jnp.transpose(out5, (2, 4, 0, 1, 3)).reshape(B, L, D)
    return out


# speedup vs baseline: 1.9167x; 1.0318x over previous
import functools

import jax
import jax.numpy as jnp
from jax import lax
from jax.experimental import pallas as pl
from jax.experimental.pallas import tpu as pltpu
from jax.experimental.pallas import tpu_sc as plsc

NC = 2
NS = 16
NW = NC * NS
LANES = 16
NBUF = 4
PREF = 2
SUBL = 8
TL = 128


def _make_kernel(B, L, V, D, C, interpret=False):
    BC = B // NW
    DT = D // SUBL
    assert BC == TL and L == C and D % LANES == 0
    assert L % NBUF == 0

    mesh = plsc.VectorSubcoreMesh(core_axis_name="c", subcore_axis_name="s")

    @functools.partial(
        pl.kernel,
        out_type=jax.ShapeDtypeStruct((L, DT, NW, SUBL * TL), jnp.float32),
        mesh=mesh,
        compiler_params=pltpu.CompilerParams(
            use_tc_tiling_on_sc=False, needs_layout_passes=False),
        interpret=interpret,
        scratch_types=[
            pltpu.VMEM((C, BC), jnp.int32),
            pltpu.VMEM((NBUF, BC, D), jnp.float32),
            pltpu.VMEM((NBUF, DT, SUBL * TL), jnp.float32),
            pltpu.VMEM((C, D), jnp.float32),
            pltpu.VMEM((2 * D,), jnp.float32),
            [pltpu.SemaphoreType.DMA] * NBUF,
            [pltpu.SemaphoreType.DMA] * NBUF,
        ],
    )
    def k(xT_hbm, tok_hbm, pos_hbm, out_hbm,
          idx_v, rows_v, outT_v, pos_v, posd_v, gsems, wsems):
        wid = lax.axis_index("s") * NC + lax.axis_index("c")
        b0 = wid * BC

        pltpu.sync_copy(xT_hbm.at[:, pl.ds(b0, BC)], idx_v)
        pltpu.sync_copy(pos_hbm, pos_v)

        def gather(lp, p):
            return pltpu.make_async_copy(
                tok_hbm.at[idx_v.at[lp]], rows_v.at[p], gsems[p])

        def writeback(lp, p):
            return pltpu.make_async_copy(
                outT_v.at[p], out_hbm.at[lp, :, wid, :], wsems[p])

        for j in range(PREF):
            gather(j, j).start()

        iota = lax.broadcasted_iota(jnp.int32, (LANES,), 0)
        rb = [iota + bs * LANES for bs in range(BC // LANES)]
        rbD = [(iota + bs * LANES) * D for bs in range(BC // LANES)]
        csts = [jnp.full((LANES,), v, jnp.int32) for v in range(NBUF)]
        ones = jnp.ones((LANES,), jnp.bool_)

        def step(l2, j):
            lp = NBUF * l2 + j
            p = j
            q = (j + PREF) % NBUF

            gather(lp, p).wait()

            for c in range(D // LANES):
                pc = pos_v[lp, pl.ds(c * LANES, LANES)]
                posd_v[pl.ds(c * LANES, LANES)] = pc
                posd_v[pl.ds(D + c * LANES, LANES)] = pc

            @plsc.parallel_loop(0, D // LANES, step=1, unroll=2)
            def dgstep(dg):
                for dj in range(LANES):
                    dstart = dg * LANES + dj
                    dvec = (dstart + iota) & (D - 1)
                    pv = posd_v[pl.ds(dstart, LANES)]
                    dtv = dvec >> 3
                    tbase = (dvec & 7) << 7
                    for bs in range(BC // LANES):
                        col = plsc.load_gather(
                            rows_v, [csts[p], rb[bs], dvec], mask=ones)
                        plsc.store_scatter(
                            outT_v,
                            [csts[p], dtv, tbase + rb[bs]],
                            col + pv, mask=ones)

            writeback(lp, p).start()

            if j < PREF:
                @pl.when(l2 >= 1)
                def _():
                    writeback(lp - PREF, q).wait()

                gather(lp + PREF, q).start()
            else:
                writeback(lp - PREF, q).wait()

                @pl.when(l2 < L // NBUF - 1)
                def _():
                    gather(lp + PREF, q).start()

        def round4(l2, carry):
            for j in range(NBUF):
                step(l2, j)
            return carry

        lax.fori_loop(0, L // NBUF, round4, 0)

        for lp in range(L - PREF, L):
            writeback(lp, lp % NBUF).wait()

    return k


def kernel(x, tok_table, pos_table):
    B, L = x.shape
    V, D = tok_table.shape
    C = pos_table.shape[0]
    k = _make_kernel(B, L, V, D, C)
    out4 = k(x.T, tok_table, pos_table)
    out5 = out4.reshape(L, D // SUBL, B // TL, SUBL, TL)
    out = jnp.transpose(out5, (2, 4, 0, 1, 3)).reshape(B, L, D)
    return out


# speedup vs baseline: 2.5852x; 1.3488x over previous
import functools

import jax
import jax.numpy as jnp
from jax import lax
from jax.experimental import pallas as pl
from jax.experimental.pallas import tpu as pltpu
from jax.experimental.pallas import tpu_sc as plsc

NC = 2
NS = 16
NW = NC * NS
LANES = 16
NBUF = 4
PREF = 2
SUBL = 8
TL = 128


def _make_kernel(B, L, V, D, C, interpret=False):
    BC = B // NW
    DT = D // SUBL
    assert BC == TL and L == C and D % LANES == 0
    assert L % NBUF == 0

    mesh = plsc.VectorSubcoreMesh(core_axis_name="c", subcore_axis_name="s")

    @functools.partial(
        pl.kernel,
        out_type=jax.ShapeDtypeStruct((L, DT, NW, SUBL * TL), jnp.float32),
        mesh=mesh,
        compiler_params=pltpu.CompilerParams(
            use_tc_tiling_on_sc=False, needs_layout_passes=False,
            disable_bounds_checks=True),
        interpret=interpret,
        scratch_types=[
            pltpu.VMEM((C, BC), jnp.int32),
            pltpu.VMEM((NBUF, BC, D), jnp.float32),
            pltpu.VMEM((NBUF, DT, SUBL * TL), jnp.float32),
            pltpu.VMEM((C, D), jnp.float32),
            pltpu.VMEM((2 * D,), jnp.float32),
            [pltpu.SemaphoreType.DMA] * NBUF,
            [pltpu.SemaphoreType.DMA] * NBUF,
        ],
    )
    def k(xT_hbm, tok_hbm, pos_hbm, out_hbm,
          idx_v, rows_v, outT_v, pos_v, posd_v, gsems, wsems):
        wid = lax.axis_index("s") * NC + lax.axis_index("c")
        b0 = wid * BC

        pltpu.sync_copy(xT_hbm.at[:, pl.ds(b0, BC)], idx_v)
        pltpu.sync_copy(pos_hbm, pos_v)

        def gather(lp, p):
            return pltpu.make_async_copy(
                tok_hbm.at[idx_v.at[lp]], rows_v.at[p], gsems[p])

        def writeback(lp, p):
            return pltpu.make_async_copy(
                outT_v.at[p], out_hbm.at[lp, :, wid, :], wsems[p])

        for j in range(PREF):
            gather(j, j).start()

        iota = lax.broadcasted_iota(jnp.int32, (LANES,), 0)
        rb = [iota + bs * LANES for bs in range(BC // LANES)]
        rbD = [(iota + bs * LANES) * D for bs in range(BC // LANES)]
        csts = [jnp.full((LANES,), v, jnp.int32) for v in range(NBUF)]
        ones = jnp.ones((LANES,), jnp.bool_)

        def step(l2, j):
            lp = NBUF * l2 + j
            p = j
            q = (j + PREF) % NBUF

            gather(lp, p).wait()

            for c in range(D // LANES):
                pc = pos_v[lp, pl.ds(c * LANES, LANES)]
                posd_v[pl.ds(c * LANES, LANES)] = pc
                posd_v[pl.ds(D + c * LANES, LANES)] = pc

            @plsc.parallel_loop(0, D, step=1, unroll=4)
            def dstep(dstart):
                dvec = (dstart + iota) & (D - 1)
                pv = posd_v[pl.ds(dstart, LANES)]
                dtv = dvec >> 3
                tbase = (dvec & 7) << 7
                cols = [
                    plsc.load_gather(
                        rows_v, [csts[p], rb[bs], dvec], mask=ones)
                    for bs in range(BC // LANES)
                ]
                for bs in range(BC // LANES):
                    plsc.store_scatter(
                        outT_v,
                        [csts[p], dtv, tbase + rb[bs]],
                        cols[bs] + pv, mask=ones)

            writeback(lp, p).start()

            if j < PREF:
                @pl.when(l2 >= 1)
                def _():
                    writeback(lp - PREF, q).wait()

                gather(lp + PREF, q).start()
            else:
                writeback(lp - PREF, q).wait()

                @pl.when(l2 < L // NBUF - 1)
                def _():
                    gather(lp + PREF, q).start()

        def round4(l2, carry):
            for j in range(NBUF):
                step(l2, j)
            return carry

        lax.fori_loop(0, L // NBUF, round4, 0)

        for lp in range(L - PREF, L):
            writeback(lp, lp % NBUF).wait()

    return k


def kernel(x, tok_table, pos_table):
    B, L = x.shape
    V, D = tok_table.shape
    C = pos_table.shape[0]
    k = _make_kernel(B, L, V, D, C)
    out4 = k(x.T, tok_table, pos_table)
    out5 = out4.reshape(L, D // SUBL, B // TL, SUBL, TL)
    out = jnp.transpose(out5, (2, 4, 0, 1, 3)).reshape(B, L, D)
    return out
